# baseline (speedup 1.0000x reference)
import jax
import jax.numpy as jnp
from jax import lax
from jax.experimental import pallas as pl
from jax.experimental.pallas import tpu as pltpu
from jax.experimental.pallas import tpu_sc as plsc

OUT_H = 224
OUT_W = 224
N = OUT_H * OUT_W
B = 4
C = 96
H = 384
W = 384
HW = H * W

NW = 32
S_PER_W = (B * N) // NW
CHUNK = 128
NCHUNK = S_PER_W // CHUNK
W_PER_B = N // S_PER_W
LANES = 16
GROUPS = CHUNK // LANES
C4 = 4 * C


BROWS = (B * HW) // NW
BBLK = 128
NBBLK = BROWS // BBLK
PAD = 512
_OFFS = (0, 1, W, W + 1)


def _build_body(tblp, table4, stag, sem):
  wid = lax.axis_index("s") * 2 + lax.axis_index("c")
  base = wid * BROWS

  def fire(bi, s):
    r0 = base + bi * BBLK
    for k, off in enumerate(_OFFS):
      pltpu.async_copy(
          tblp.at[pl.ds(r0 + off, BBLK), :],
          stag.at[s, :, pl.ds(k * C, C)],
          sem,
      )

  fire(0, 0)

  def blk_body(bi, _):
    s = bi & 1
    r0 = base + bi * BBLK
    for k, off in enumerate(_OFFS):
      pltpu.make_async_copy(
          tblp.at[pl.ds(r0 + off, BBLK), :],
          stag.at[s, :, pl.ds(k * C, C)],
          sem,
      ).wait()

    @pl.when(bi + 1 < NBBLK)
    def _():
      fire(bi + 1, 1 - s)

    pltpu.sync_copy(stag.at[s], table4.at[pl.ds(r0, BBLK), :])
    return 0

  lax.fori_loop(0, NBBLK, blk_body, 0)


def _sc_body(table, xs, ys, thetab, out,
             xsv, ysv, thv, idx_v,
             w_a, w_b, w_c, w_d,
             rows, outv, sem, sem_out):
  wid = lax.axis_index("s") * 2 + lax.axis_index("c")
  bb = wid // W_PER_B
  nb = (wid % W_PER_B) * S_PER_W
  base_row = bb * HW
  gb0 = bb * N + nb

  pltpu.sync_copy(thetab.at[bb], thv)
  t0 = thv[0, :]
  t1 = thv[1, :]
  t2 = thv[2, :]
  t3 = thv[3, :]
  t4 = thv[4, :]
  t5 = thv[5, :]

  zero = jnp.zeros((LANES,), jnp.float32)

  def coords_and_fire(ci, s):
    nbase = nb + ci * CHUNK
    pltpu.sync_copy(xs.at[pl.ds(nbase, CHUNK)], xsv)
    pltpu.sync_copy(ys.at[pl.ds(nbase, CHUNK)], ysv)

    def coord_body(g, _):
      gs = g * LANES
      xg = xsv[pl.ds(gs, LANES)]
      yg = ysv[pl.ds(gs, LANES)]
      px = (t0 * xg + t1 * yg + t2 + 1.0) * (0.5 * W)
      py = (t3 * xg + t4 * yg + t5 + 1.0) * (0.5 * H)
      xt = px.astype(jnp.int32)
      x0 = jnp.where(xt.astype(jnp.float32) > px, xt - 1, xt)
      yt = py.astype(jnp.int32)
      y0 = jnp.where(yt.astype(jnp.float32) > py, yt - 1, yt)
      x0c = jnp.clip(x0, 0, W - 1)
      x1c = jnp.clip(x0 + 1, 0, W - 1)
      y0c = jnp.clip(y0, 0, H - 1)
      y1c = jnp.clip(y0 + 1, 0, H - 1)
      idx_v[pl.ds(gs, LANES)] = y0c * W + x0c + base_row
      x0f = x0c.astype(jnp.float32)
      x1f = x1c.astype(jnp.float32)
      y0f = y0c.astype(jnp.float32)
      y1f = y1c.astype(jnp.float32)
      wa = (x1f - px) * (y1f - py)
      wb = (x1f - px) * (py - y0f)
      wc = (px - x0f) * (y1f - py)
      wd = (px - x0f) * (py - y0f)
      xeq = x1c == x0c
      yeq = y1c == y0c
      wa = wa + jnp.where(xeq, wc, zero)
      wc = jnp.where(xeq, zero, wc)
      wb = wb + jnp.where(xeq, wd, zero)
      wd = jnp.where(xeq, zero, wd)
      wa = wa + jnp.where(yeq, wb, zero)
      wb = jnp.where(yeq, zero, wb)
      wc = wc + jnp.where(yeq, wd, zero)
      wd = jnp.where(yeq, zero, wd)
      w_a[s, pl.ds(gs, LANES)] = wa
      w_b[s, pl.ds(gs, LANES)] = wb
      w_c[s, pl.ds(gs, LANES)] = wc
      w_d[s, pl.ds(gs, LANES)] = wd
      return 0

    lax.fori_loop(0, GROUPS, coord_body, 0)
    pltpu.async_copy(table.at[idx_v], rows.at[s], sem)

  coords_and_fire(0, 0)

  def chunk_body(ci, _):
    s = ci & 1
    sn = 1 - s
    gbase = gb0 + ci * CHUNK

    pltpu.make_async_copy(table.at[idx_v], rows.at[s], sem).wait()

    @pl.when(ci + 1 < NCHUNK)
    def _():
      coords_and_fire(ci + 1, sn)

    @pl.when(ci >= 2)
    def _():
      pltpu.make_async_copy(
          outv.at[s], out.at[pl.ds(gbase, CHUNK), :], sem_out
      ).wait()

    sv = jnp.full((LANES,), s, jnp.int32)

    @plsc.parallel_loop(0, CHUNK, step=1, unroll=4)
    def sample_body(smp):
      smpv = jnp.full((LANES,), smp, jnp.int32)
      wav = plsc.load_gather(w_a, [sv, smpv])
      wbv = plsc.load_gather(w_b, [sv, smpv])
      wcv = plsc.load_gather(w_c, [sv, smpv])
      wdv = plsc.load_gather(w_d, [sv, smpv])
      for cb in range(C // LANES):
        co = cb * LANES
        va = rows[s, smp, pl.ds(co, LANES)]
        vc = rows[s, smp, pl.ds(co + C, LANES)]
        vb = rows[s, smp, pl.ds(co + 2 * C, LANES)]
        vd = rows[s, smp, pl.ds(co + 3 * C, LANES)]
        acc = ((wav * va + wbv * vb) + wcv * vc) + wdv * vd
        outv[s, smp, pl.ds(co, LANES)] = acc

    pltpu.async_copy(outv.at[s], out.at[pl.ds(gbase, CHUNK), :], sem_out)
    return 0

  lax.fori_loop(0, NCHUNK, chunk_body, 0)

  pltpu.make_async_copy(
      outv.at[0], out.at[pl.ds(gb0, CHUNK), :], sem_out
  ).wait()
  pltpu.make_async_copy(
      outv.at[1], out.at[pl.ds(gb0, CHUNK), :], sem_out
  ).wait()


@jax.jit
def kernel(X, affine_transformation):
  tblp = jnp.pad(
      jnp.transpose(X, (0, 2, 3, 1)).reshape(B * HW, C), ((0, PAD), (0, 0))
  )

  mesh = plsc.VectorSubcoreMesh(core_axis_name="c", subcore_axis_name="s")
  build = pl.kernel(
      _build_body,
      out_type=jax.ShapeDtypeStruct((B * HW, C4), jnp.float32),
      mesh=mesh,
      compiler_params=pltpu.CompilerParams(
          needs_layout_passes=False, use_tc_tiling_on_sc=False
      ),
      scratch_types=[
          pltpu.VMEM((2, BBLK, C4), jnp.float32),
          pltpu.SemaphoreType.DMA,
      ],
  )
  table = build(tblp)
  thetab = jnp.broadcast_to(
      lax.reduce_precision(
          affine_transformation.astype(jnp.float32), 8, 7
      ).reshape(B, 6, 1),
      (B, 6, LANES),
  )

  x_lin = jnp.linspace(-1.0, 1.0, OUT_W, dtype=jnp.float32)
  y_lin = jnp.linspace(-1.0, 1.0, OUT_H, dtype=jnp.float32)
  xc, yc = jnp.meshgrid(x_lin, y_lin, indexing="ij")
  xs = lax.reduce_precision(xc.reshape(-1), 8, 7)
  ys = lax.reduce_precision(yc.reshape(-1), 8, 7)

  mesh = plsc.VectorSubcoreMesh(core_axis_name="c", subcore_axis_name="s")
  grid_sample = pl.kernel(
      _sc_body,
      out_type=jax.ShapeDtypeStruct((B * N, C), jnp.float32),
      mesh=mesh,
      compiler_params=pltpu.CompilerParams(
          needs_layout_passes=False, use_tc_tiling_on_sc=False
      ),
      scratch_types=[
          pltpu.VMEM((CHUNK,), jnp.float32),
          pltpu.VMEM((CHUNK,), jnp.float32),
          pltpu.VMEM((6, LANES), jnp.float32),
          pltpu.VMEM((CHUNK,), jnp.int32),
          pltpu.VMEM((2, CHUNK), jnp.float32),
          pltpu.VMEM((2, CHUNK), jnp.float32),
          pltpu.VMEM((2, CHUNK), jnp.float32),
          pltpu.VMEM((2, CHUNK), jnp.float32),
          pltpu.VMEM((2, CHUNK, C4), jnp.float32),
          pltpu.VMEM((2, CHUNK, C), jnp.float32),
          pltpu.SemaphoreType.DMA,
          pltpu.SemaphoreType.DMA,
      ],
  )
  out2 = grid_sample(table, xs, ys, thetab)
  return jnp.transpose(out2.reshape(B, N, C), (0, 2, 1))


# speedup vs baseline: 1.0367x; 1.0367x over previous
import jax
import jax.numpy as jnp
from jax import lax
from jax.experimental import pallas as pl
from jax.experimental.pallas import tpu as pltpu
from jax.experimental.pallas import tpu_sc as plsc

OUT_H = 224
OUT_W = 224
N = OUT_H * OUT_W
B = 4
C = 96
H = 384
W = 384
HW = H * W

NW = 32
S_PER_W = (B * N) // NW
CHUNK = 128
NCHUNK = S_PER_W // CHUNK
W_PER_B = N // S_PER_W
LANES = 16
GROUPS = CHUNK // LANES
C4 = 4 * C


BROWS = (B * HW) // NW
BBLK = 128
NBBLK = BROWS // BBLK
PAD = 512
_OFFS = (0, 1, W, W + 1)


def _build_body(tblp, table4, stag, sem):
  wid = lax.axis_index("s") * 2 + lax.axis_index("c")
  base = wid * BROWS

  def fire(bi, s):
    r0 = base + bi * BBLK
    for k, off in enumerate(_OFFS):
      pltpu.async_copy(
          tblp.at[pl.ds(r0 + off, BBLK), :],
          stag.at[s, :, pl.ds(k * C, C)],
          sem,
      )

  fire(0, 0)

  def blk_body(bi, _):
    s = bi & 1
    r0 = base + bi * BBLK
    for k, off in enumerate(_OFFS):
      pltpu.make_async_copy(
          tblp.at[pl.ds(r0 + off, BBLK), :],
          stag.at[s, :, pl.ds(k * C, C)],
          sem,
      ).wait()

    @pl.when(bi + 1 < NBBLK)
    def _():
      fire(bi + 1, 1 - s)

    pltpu.sync_copy(stag.at[s], table4.at[pl.ds(r0, BBLK), :])
    return 0

  lax.fori_loop(0, NBBLK, blk_body, 0)


def _sc_body(table, xs, ys, thetab, out,
             xsv, ysv, thv, idx_v,
             w_a, w_b, w_c, w_d,
             rows, outv, sem, sem_out):
  wid = lax.axis_index("s") * 2 + lax.axis_index("c")
  bb = wid // W_PER_B
  nb = (wid % W_PER_B) * S_PER_W
  base_row = bb * HW
  gb0 = bb * N + nb

  pltpu.sync_copy(thetab.at[bb], thv)
  t0 = thv[0, :]
  t1 = thv[1, :]
  t2 = thv[2, :]
  t3 = thv[3, :]
  t4 = thv[4, :]
  t5 = thv[5, :]

  lane = lax.iota(jnp.int32, LANES)
  zero = jnp.zeros((LANES,), jnp.float32)

  def coords_and_fire(ci, s):
    nbase = nb + ci * CHUNK
    pltpu.sync_copy(xs.at[pl.ds(nbase, CHUNK)], xsv)
    pltpu.sync_copy(ys.at[pl.ds(nbase, CHUNK)], ysv)

    def coord_body(g, _):
      gs = g * LANES
      xg = xsv[pl.ds(gs, LANES)]
      yg = ysv[pl.ds(gs, LANES)]
      px = (t0 * xg + t1 * yg + t2 + 1.0) * (0.5 * W)
      py = (t3 * xg + t4 * yg + t5 + 1.0) * (0.5 * H)
      xt = px.astype(jnp.int32)
      x0 = jnp.where(xt.astype(jnp.float32) > px, xt - 1, xt)
      yt = py.astype(jnp.int32)
      y0 = jnp.where(yt.astype(jnp.float32) > py, yt - 1, yt)
      x0c = jnp.clip(x0, 0, W - 1)
      x1c = jnp.clip(x0 + 1, 0, W - 1)
      y0c = jnp.clip(y0, 0, H - 1)
      y1c = jnp.clip(y0 + 1, 0, H - 1)
      idx_v[pl.ds(gs, LANES)] = y0c * W + x0c + base_row
      x0f = x0c.astype(jnp.float32)
      x1f = x1c.astype(jnp.float32)
      y0f = y0c.astype(jnp.float32)
      y1f = y1c.astype(jnp.float32)
      wa = (x1f - px) * (y1f - py)
      wb = (x1f - px) * (py - y0f)
      wc = (px - x0f) * (y1f - py)
      wd = (px - x0f) * (py - y0f)
      xeq = x1c == x0c
      yeq = y1c == y0c
      wa = wa + jnp.where(xeq, wc, zero)
      wc = jnp.where(xeq, zero, wc)
      wb = wb + jnp.where(xeq, wd, zero)
      wd = jnp.where(xeq, zero, wd)
      wa = wa + jnp.where(yeq, wb, zero)
      wb = jnp.where(yeq, zero, wb)
      wc = wc + jnp.where(yeq, wd, zero)
      wd = jnp.where(yeq, zero, wd)
      w_a[s, pl.ds(gs, LANES)] = wa
      w_b[s, pl.ds(gs, LANES)] = wb
      w_c[s, pl.ds(gs, LANES)] = wc
      w_d[s, pl.ds(gs, LANES)] = wd
      return 0

    lax.fori_loop(0, GROUPS, coord_body, 0)
    pltpu.async_copy(table.at[idx_v], rows.at[s], sem)

  coords_and_fire(0, 0)

  def chunk_body(ci, _):
    s = ci & 1
    sn = 1 - s
    gbase = gb0 + ci * CHUNK

    pltpu.make_async_copy(table.at[idx_v], rows.at[s], sem).wait()

    @pl.when(ci + 1 < NCHUNK)
    def _():
      coords_and_fire(ci + 1, sn)

    @pl.when(ci >= 2)
    def _():
      pltpu.make_async_copy(
          outv.at[s], out.at[bb, :, pl.ds(nb, CHUNK)], sem_out
      ).wait()

    sv = jnp.full((LANES,), s, jnp.int32)
    cols = [cb * LANES + lane for cb in range(C // LANES)]

    @plsc.parallel_loop(0, CHUNK, step=1, unroll=4)
    def sample_body(smp):
      smpv = jnp.full((LANES,), smp, jnp.int32)
      wav = plsc.load_gather(w_a, [sv, smpv])
      wbv = plsc.load_gather(w_b, [sv, smpv])
      wcv = plsc.load_gather(w_c, [sv, smpv])
      wdv = plsc.load_gather(w_d, [sv, smpv])
      for cb in range(C // LANES):
        co = cb * LANES
        va = rows[s, smp, pl.ds(co, LANES)]
        vc = rows[s, smp, pl.ds(co + C, LANES)]
        vb = rows[s, smp, pl.ds(co + 2 * C, LANES)]
        vd = rows[s, smp, pl.ds(co + 3 * C, LANES)]
        acc = ((wav * va + wbv * vb) + wcv * vc) + wdv * vd
        plsc.store_scatter(outv, [sv, cols[cb], smpv], acc)

    nbase = nb + ci * CHUNK
    pltpu.async_copy(outv.at[s], out.at[bb, :, pl.ds(nbase, CHUNK)], sem_out)
    return 0

  lax.fori_loop(0, NCHUNK, chunk_body, 0)

  pltpu.make_async_copy(
      outv.at[0], out.at[bb, :, pl.ds(nb, CHUNK)], sem_out
  ).wait()
  pltpu.make_async_copy(
      outv.at[1], out.at[bb, :, pl.ds(nb, CHUNK)], sem_out
  ).wait()


@jax.jit
def kernel(X, affine_transformation):
  tblp = jnp.pad(
      jnp.transpose(X, (0, 2, 3, 1)).reshape(B * HW, C), ((0, PAD), (0, 0))
  )

  mesh = plsc.VectorSubcoreMesh(core_axis_name="c", subcore_axis_name="s")
  build = pl.kernel(
      _build_body,
      out_type=jax.ShapeDtypeStruct((B * HW, C4), jnp.float32),
      mesh=mesh,
      compiler_params=pltpu.CompilerParams(
          needs_layout_passes=False, use_tc_tiling_on_sc=False
      ),
      scratch_types=[
          pltpu.VMEM((2, BBLK, C4), jnp.float32),
          pltpu.SemaphoreType.DMA,
      ],
  )
  table = build(tblp)
  thetab = jnp.broadcast_to(
      lax.reduce_precision(
          affine_transformation.astype(jnp.float32), 8, 7
      ).reshape(B, 6, 1),
      (B, 6, LANES),
  )

  x_lin = jnp.linspace(-1.0, 1.0, OUT_W, dtype=jnp.float32)
  y_lin = jnp.linspace(-1.0, 1.0, OUT_H, dtype=jnp.float32)
  xc, yc = jnp.meshgrid(x_lin, y_lin, indexing="ij")
  xs = lax.reduce_precision(xc.reshape(-1), 8, 7)
  ys = lax.reduce_precision(yc.reshape(-1), 8, 7)

  mesh = plsc.VectorSubcoreMesh(core_axis_name="c", subcore_axis_name="s")
  grid_sample = pl.kernel(
      _sc_body,
      out_type=jax.ShapeDtypeStruct((B, C, N), jnp.float32),
      mesh=mesh,
      compiler_params=pltpu.CompilerParams(
          needs_layout_passes=False, use_tc_tiling_on_sc=False
      ),
      scratch_types=[
          pltpu.VMEM((CHUNK,), jnp.float32),
          pltpu.VMEM((CHUNK,), jnp.float32),
          pltpu.VMEM((6, LANES), jnp.float32),
          pltpu.VMEM((CHUNK,), jnp.int32),
          pltpu.VMEM((2, CHUNK), jnp.float32),
          pltpu.VMEM((2, CHUNK), jnp.float32),
          pltpu.VMEM((2, CHUNK), jnp.float32),
          pltpu.VMEM((2, CHUNK), jnp.float32),
          pltpu.VMEM((2, CHUNK, C4), jnp.float32),
          pltpu.VMEM((2, C, CHUNK), jnp.float32),
          pltpu.SemaphoreType.DMA,
          pltpu.SemaphoreType.DMA,
      ],
  )
  return grid_sample(table, xs, ys, thetab)
